# baseline (speedup 1.0000x reference)
import functools

import jax
import jax.numpy as jnp
from jax import lax
from jax.experimental import pallas as pl
from jax.experimental.pallas import tpu as pltpu
from jax.experimental.pallas import tpu_sc as plsc


def kernel(x, pe):
    S, D = pe.shape
    info = plsc.get_sparse_core_info()
    nc, ns = info.num_cores, info.num_subcores
    nw = nc * ns
    rows = S // nw

    mesh = plsc.VectorSubcoreMesh(core_axis_name="c", subcore_axis_name="s")

    @functools.partial(
        pl.kernel,
        mesh=mesh,
        out_type=jax.ShapeDtypeStruct((S, D), pe.dtype),
    )
    def sc_copy(pe_hbm, out_hbm):
        wid = lax.axis_index("s") * nc + lax.axis_index("c")
        base = wid * rows
        pltpu.sync_copy(
            pe_hbm.at[pl.ds(base, rows)],
            out_hbm.at[pl.ds(base, rows)],
        )

    return sc_copy(pe)[None, :, :]


# speedup vs baseline: 24.3855x; 24.3855x over previous
import functools

import jax
import jax.numpy as jnp
from jax import lax
from jax.experimental import pallas as pl
from jax.experimental.pallas import tpu as pltpu
from jax.experimental.pallas import tpu_sc as plsc


def kernel(x, pe):
    S, D = pe.shape
    info = plsc.get_sparse_core_info()
    nc, ns = info.num_cores, info.num_subcores
    nw = nc * ns
    rows = S // nw
    CH = 32
    NCH = rows // CH

    mesh = plsc.VectorSubcoreMesh(core_axis_name="c", subcore_axis_name="s")

    @functools.partial(
        pl.kernel,
        mesh=mesh,
        out_type=jax.ShapeDtypeStruct((S, D), pe.dtype),
        scratch_types=[
            pltpu.VMEM((CH, D), jnp.float32),
            pltpu.VMEM((CH, D), jnp.float32),
            pltpu.SemaphoreType.DMA,
            pltpu.SemaphoreType.DMA,
            pltpu.SemaphoreType.DMA,
            pltpu.SemaphoreType.DMA,
        ],
    )
    def sc_copy(pe_hbm, out_hbm, buf0, buf1, si0, si1, so0, so1):
        wid = lax.axis_index("s") * nc + lax.axis_index("c")
        base = wid * rows
        bufs = [buf0, buf1]
        in_sems = [si0, si1]
        out_sems = [so0, so1]
        in_copies = [None, None]
        out_copies = [None, None]

        in_copies[0] = pltpu.async_copy(
            pe_hbm.at[pl.ds(base, CH)], bufs[0], in_sems[0])
        for c in range(NCH):
            b = c % 2
            nb = (c + 1) % 2
            if c + 1 < NCH:
                if out_copies[nb] is not None:
                    out_copies[nb].wait()
                in_copies[nb] = pltpu.async_copy(
                    pe_hbm.at[pl.ds(base + (c + 1) * CH, CH)],
                    bufs[nb], in_sems[nb])
            in_copies[b].wait()
            out_copies[b] = pltpu.async_copy(
                bufs[b],
                out_hbm.at[pl.ds(base + c * CH, CH)],
                out_sems[b])
        for b in range(2):
            if out_copies[b] is not None:
                out_copies[b].wait()

    return sc_copy(pe)[None, :, :]
